# baseline (speedup 1.0000x reference)
import functools

import jax
import jax.numpy as jnp
from jax import lax
from jax.experimental import pallas as pl
from jax.experimental.pallas import tpu as pltpu
from jax.experimental.pallas import tpu_sc as plsc

_B, _C, _H, _W = 8, 64, 64, 64
_E, _K = 5, 3
_HW = _H * _W
_C2 = 2 * _C


def _ln_norm(x):
    onesc = jnp.full((_C, 1), 1.0 / _C, jnp.float32)
    mu = jnp.dot(x, onesc, preferred_element_type=jnp.float32)
    xcb = (x - mu).astype(jnp.bfloat16)
    onesb = jnp.full((_C, 1), 1.0 / _C, jnp.bfloat16)
    var = jnp.dot(xcb * xcb, onesb, preferred_element_type=jnp.float32)
    rstd = lax.rsqrt(var + 1e-5)
    return xcb * rstd.astype(jnp.bfloat16)


def _dw3x3(y, wdw_ref, bdw_ref):
    y3 = y.reshape(_H, _W, _C2)
    zc = jnp.zeros((_H, 1, _C2), y.dtype)
    shifted_w = {
        -1: jnp.concatenate([zc, y3[:, :-1, :]], axis=1),
        0: y3,
        1: jnp.concatenate([y3[:, 1:, :], zc], axis=1),
    }
    zr = jnp.zeros((1, _W, _C2), y.dtype)
    acc = jnp.zeros((_H, _W, _C2), y.dtype)
    t = 0
    for i in (-1, 0, 1):
        for j in (-1, 0, 1):
            v = shifted_w[j]
            if i == -1:
                v = jnp.concatenate([zr, v[:-1]], axis=0)
            elif i == 1:
                v = jnp.concatenate([v[1:], zr], axis=0)
            acc = acc + v * wdw_ref[0, t]
            t += 1
    return (acc + bdw_ref[0]).reshape(_HW, _C2)


def _naf_one(x, xn, w1_ref, bb1_ref, wdw_ref, bdw_ref, wsca_ref, bsca_ref,
             w3_ref, b3_ref, w4_ref, bb4_ref, w5_ref, b5_ref,
             beta_ref, gamma_ref):
    y = jnp.dot(xn, w1_ref[0],
                preferred_element_type=jnp.float32).astype(jnp.bfloat16)
    y = y + bb1_ref[0]
    y = _dw3x3(y, wdw_ref, bdw_ref)
    a = y[:, :_C] * y[:, _C:]
    ones_hw = jnp.full((1, _HW), 1.0 / _HW, jnp.bfloat16)
    s = jnp.dot(ones_hw, a, preferred_element_type=jnp.float32)
    s = jnp.dot(s, wsca_ref[0], preferred_element_type=jnp.float32) + bsca_ref[0]
    y = a * s.astype(jnp.bfloat16)
    y = jnp.dot(y, w3_ref[0], preferred_element_type=jnp.float32) + b3_ref[0]
    x2 = x + y * beta_ref[0]

    xn2 = _ln_norm(x2)
    y = jnp.dot(xn2, w4_ref[0],
                preferred_element_type=jnp.float32).astype(jnp.bfloat16)
    y = y + bb4_ref[0]
    a = y[:, :_C] * y[:, _C:]
    y = jnp.dot(a, w5_ref[0], preferred_element_type=jnp.float32) + b5_ref[0]
    return x2 + y * gamma_ref[0]


_NP = 14


def _naf_body(topk_ref, nw_ref, feat_ref, *refs):
    out_ref = refs[-1]
    b = pl.program_id(0)
    x = jnp.transpose(feat_ref[0])
    xn = _ln_norm(x)
    acc = None
    for k in range(_K):
        res = _naf_one(x, xn, *refs[k * _NP:(k + 1) * _NP])
        g = nw_ref[b, k]
        acc = g * res if acc is None else acc + g * res
    out_ref[0] = jnp.transpose(acc)


_PARAM_DIMS = [(_C, _C2), (1, _C2), (9, _C2), (1, _C2), (_C, _C), (1, _C),
               (_C, _C), (1, _C), (_C, _C2), (1, _C2), (_C, _C), (1, _C),
               (1, _C), (1, _C)]


def _run_naf(topk_i, nw, feat_r, *params, interpret=False):
    assert len(params) == _NP

    def bidx(b, topk_ref, nw_ref):
        return (b, 0, 0)

    in_specs = [pl.BlockSpec((1, _C, _HW), bidx)]
    for k in range(_K):
        def eidx(b, topk_ref, nw_ref, _k=k):
            return (topk_ref[b, _k], 0, 0)
        in_specs.extend(
            pl.BlockSpec((1,) + dims, eidx) for dims in _PARAM_DIMS)

    grid_spec = pltpu.PrefetchScalarGridSpec(
        num_scalar_prefetch=2,
        grid=(_B,),
        in_specs=in_specs,
        out_specs=pl.BlockSpec((1, _C, _HW), bidx),
    )
    return pl.pallas_call(
        _naf_body,
        grid_spec=grid_spec,
        out_shape=jax.ShapeDtypeStruct((_B, _C, _HW), jnp.float32),
        compiler_params=pltpu.CompilerParams(
            dimension_semantics=("arbitrary",)),
        interpret=interpret,
    )(topk_i, nw, feat_r, *(params * _K))


_L = 16


def _router_body(w_hbm, topk_hbm, nw_hbm, counts_hbm,
                 w_vm, topk_vm, nw_vm, counts_vm, rot_vm):
    cid = lax.axis_index("c")
    sid = lax.axis_index("s")

    @pl.when(jnp.logical_and(cid == 0, sid == 0))
    def _():
        pltpu.sync_copy(w_hbm, w_vm)
        iota = lax.iota(jnp.int32, _L)
        valid = iota < _E
        counts = jnp.zeros((_L,), jnp.int32)
        for b in range(_B):
            w = w_vm[b]
            rot_vm[pl.ds(0, _L)] = w
            rot_vm[pl.ds(_L, _L)] = w
            rank = jnp.zeros((_L,), jnp.int32)
            for r in range(1, _E):
                wr = rot_vm[pl.ds(r, _L)]
                ahead = (wr > w) | ((wr == w) & (iota >= _E - r))
                rank = rank + jnp.where(ahead, 1, 0)
            sel = (rank < _K) & valid
            counts = counts + jnp.where(sel, 1, 0)
            v = jnp.where(sel, w, 0.0)
            for sh in (8, 4, 2, 1):
                rot_vm[pl.ds(0, _L)] = v
                rot_vm[pl.ds(_L, _L)] = v
                v = v + rot_vm[pl.ds(sh, _L)]
            nwv = w / v
            prank = jnp.where(valid, rank, iota)
            plsc.store_scatter(topk_vm.at[b], [prank], iota)
            plsc.store_scatter(nw_vm.at[b], [prank], nwv)
        counts_vm[...] = counts
        pltpu.sync_copy(topk_vm, topk_hbm)
        pltpu.sync_copy(nw_vm, nw_hbm)
        pltpu.sync_copy(counts_vm, counts_hbm)


def _route_sc(weights):
    wp = jnp.concatenate(
        [weights, weights, weights, weights[:, :_L - 3 * _E]], axis=1)
    run = pl.kernel(
        _router_body,
        out_type=(jax.ShapeDtypeStruct((_B, _L), jnp.int32),
                  jax.ShapeDtypeStruct((_B, _L), jnp.float32),
                  jax.ShapeDtypeStruct((_L,), jnp.int32)),
        mesh=plsc.VectorSubcoreMesh(core_axis_name="c", subcore_axis_name="s"),
        compiler_params=pltpu.CompilerParams(needs_layout_passes=False),
        scratch_types=(pltpu.VMEM((_B, _L), jnp.float32),
                       pltpu.VMEM((_B, _L), jnp.int32),
                       pltpu.VMEM((_B, _L), jnp.float32),
                       pltpu.VMEM((_L,), jnp.int32),
                       pltpu.VMEM((2 * _L,), jnp.float32)),
    )
    topk_p, nw_p, counts_p = run(wp)
    return topk_p[:, :_K], nw_p[:, :_K], counts_p[:_E]


def kernel(feat, weights, w1, b1, wdw, bdw, wsca, bsca, w3, b3,
           ln1_g, ln1_b, w4, b4, w5, b5, ln2_g, ln2_b, beta, gamma):
    topk_i, nw, counts = _route_sc(weights)

    feat_r = feat.reshape(_B, _C, _HW)
    bf = jnp.bfloat16
    tr = lambda m: m.transpose(0, 2, 1)
    row = lambda v: v.reshape(_E, 1, -1)
    w1g = (tr(w1) * ln1_g[:, :, None]).astype(bf)
    bb1 = row(b1 + jnp.einsum('ec,eoc->eo', ln1_b, w1)).astype(bf)
    w4g = (tr(w4) * ln2_g[:, :, None]).astype(bf)
    bb4 = row(b4 + jnp.einsum('ec,eoc->eo', ln2_b, w4)).astype(bf)
    wdw9 = wdw.reshape(_E, _C2, 9).transpose(0, 2, 1).astype(bf)
    out_r = _run_naf(
        topk_i, nw, feat_r, w1g, bb1, wdw9, row(bdw).astype(bf), tr(wsca),
        row(bsca), tr(w3).astype(bf), row(b3), w4g, bb4,
        tr(w5).astype(bf), row(b5), row(beta), row(gamma))
    out = out_r.reshape(_B, _C, _H, _W)
    return (out, counts, weights)


# speedup vs baseline: 1.1324x; 1.1324x over previous
import functools

import jax
import jax.numpy as jnp
from jax import lax
from jax.experimental import pallas as pl
from jax.experimental.pallas import tpu as pltpu
from jax.experimental.pallas import tpu_sc as plsc

_B, _C, _H, _W = 8, 64, 64, 64
_E, _K = 5, 3
_HW = _H * _W
_C2 = 2 * _C


def _ln_norm(x):
    onesc = jnp.full((_C, 1), 1.0 / _C, jnp.float32)
    mu = jnp.dot(x, onesc, preferred_element_type=jnp.float32)
    xcb = (x - mu).astype(jnp.bfloat16)
    onesb = jnp.full((_C, 1), 1.0 / _C, jnp.bfloat16)
    var = jnp.dot(xcb * xcb, onesb, preferred_element_type=jnp.float32)
    rstd = lax.rsqrt(var + 1e-5)
    return xcb * rstd.astype(jnp.bfloat16)


def _dw3x3(y, wdw_ref, bdw_ref):
    y3 = y.reshape(_H, _W, _C2)
    zc = jnp.zeros((_H, 1, _C2), y.dtype)
    shifted_w = {
        -1: jnp.concatenate([zc, y3[:, :-1, :]], axis=1),
        0: y3,
        1: jnp.concatenate([y3[:, 1:, :], zc], axis=1),
    }
    zr = jnp.zeros((1, _W, _C2), y.dtype)
    acc = jnp.zeros((_H, _W, _C2), y.dtype)
    t = 0
    for i in (-1, 0, 1):
        for j in (-1, 0, 1):
            v = shifted_w[j]
            if i == -1:
                v = jnp.concatenate([zr, v[:-1]], axis=0)
            elif i == 1:
                v = jnp.concatenate([v[1:], zr], axis=0)
            acc = acc + v * wdw_ref[0, t]
            t += 1
    return (acc + bdw_ref[0]).reshape(_HW, _C2)


def _naf_one(x, xn, w1_ref, bb1_ref, wdw_ref, bdw_ref, wsca_ref, bsca_ref,
             w3_ref, b3_ref, w4_ref, bb4_ref, w5_ref, b5_ref,
             beta_ref, gamma_ref):
    y = jnp.dot(xn, w1_ref[0],
                preferred_element_type=jnp.float32).astype(jnp.bfloat16)
    y = y + bb1_ref[0]
    y = _dw3x3(y, wdw_ref, bdw_ref)
    a = y[:, :_C] * y[:, _C:]
    ones_hw = jnp.full((1, _HW), 1.0 / _HW, jnp.bfloat16)
    s = jnp.dot(ones_hw, a, preferred_element_type=jnp.float32)
    s = jnp.dot(s, wsca_ref[0], preferred_element_type=jnp.float32) + bsca_ref[0]
    y = a * s.astype(jnp.bfloat16)
    y = jnp.dot(y, w3_ref[0], preferred_element_type=jnp.float32) + b3_ref[0]
    x2 = x + y * beta_ref[0]

    xn2 = _ln_norm(x2)
    y = jnp.dot(xn2, w4_ref[0],
                preferred_element_type=jnp.float32).astype(jnp.bfloat16)
    y = y + bb4_ref[0]
    a = y[:, :_C] * y[:, _C:]
    y = jnp.dot(a, w5_ref[0], preferred_element_type=jnp.float32) + b5_ref[0]
    return x2 + y * gamma_ref[0]


_NP = 14


_IMGS = 2


def _naf_body(topk_ref, nw_ref, feat_ref, *refs):
    out_ref = refs[-1]
    b = pl.program_id(0)
    for i in range(_IMGS):
        x = feat_ref[i]
        xn = _ln_norm(x)
        acc = None
        for k in range(_K):
            kk = i * _K + k
            res = _naf_one(x, xn, *refs[kk * _NP:(kk + 1) * _NP])
            g = nw_ref[b * _IMGS + i, k]
            acc = g * res if acc is None else acc + g * res
        out_ref[i] = acc


_PARAM_DIMS = [(_C, _C2), (1, _C2), (9, _C2), (1, _C2), (_C, _C), (1, _C),
               (_C, _C), (1, _C), (_C, _C2), (1, _C2), (_C, _C), (1, _C),
               (1, _C), (1, _C)]


def _run_naf(topk_i, nw, feat_r, *params, interpret=False):
    assert len(params) == _NP

    def bidx(b, topk_ref, nw_ref):
        return (b, 0, 0)

    in_specs = [pl.BlockSpec((_IMGS, _HW, _C), bidx)]
    for i in range(_IMGS):
        for k in range(_K):
            def eidx(b, topk_ref, nw_ref, _i=i, _k=k):
                return (topk_ref[b * _IMGS + _i, _k], 0, 0)
            in_specs.extend(
                pl.BlockSpec((1,) + dims, eidx) for dims in _PARAM_DIMS)

    grid_spec = pltpu.PrefetchScalarGridSpec(
        num_scalar_prefetch=2,
        grid=(_B // _IMGS,),
        in_specs=in_specs,
        out_specs=pl.BlockSpec((_IMGS, _HW, _C), bidx),
    )
    return pl.pallas_call(
        _naf_body,
        grid_spec=grid_spec,
        out_shape=jax.ShapeDtypeStruct((_B, _HW, _C), jnp.float32),
        compiler_params=pltpu.CompilerParams(
            dimension_semantics=("arbitrary",)),
        interpret=interpret,
    )(topk_i, nw, feat_r, *(params * (_K * _IMGS)))


_L = 16


def _router_body(w_hbm, topk_hbm, nw_hbm, counts_hbm,
                 w_vm, topk_vm, nw_vm, counts_vm, rot_vm):
    cid = lax.axis_index("c")
    sid = lax.axis_index("s")

    @pl.when(jnp.logical_and(cid == 0, sid == 0))
    def _():
        pltpu.sync_copy(w_hbm, w_vm)
        iota = lax.iota(jnp.int32, _L)
        valid = iota < _E
        counts = jnp.zeros((_L,), jnp.int32)
        for b in range(_B):
            w = w_vm[b]
            rot_vm[pl.ds(0, _L)] = w
            rot_vm[pl.ds(_L, _L)] = w
            rank = jnp.zeros((_L,), jnp.int32)
            for r in range(1, _E):
                wr = rot_vm[pl.ds(r, _L)]
                ahead = (wr > w) | ((wr == w) & (iota >= _E - r))
                rank = rank + jnp.where(ahead, 1, 0)
            sel = (rank < _K) & valid
            counts = counts + jnp.where(sel, 1, 0)
            v = jnp.where(sel, w, 0.0)
            for sh in (8, 4, 2, 1):
                rot_vm[pl.ds(0, _L)] = v
                rot_vm[pl.ds(_L, _L)] = v
                v = v + rot_vm[pl.ds(sh, _L)]
            nwv = w / v
            prank = jnp.where(valid, rank, iota)
            plsc.store_scatter(topk_vm.at[b], [prank], iota)
            plsc.store_scatter(nw_vm.at[b], [prank], nwv)
        counts_vm[...] = counts
        pltpu.sync_copy(topk_vm, topk_hbm)
        pltpu.sync_copy(nw_vm, nw_hbm)
        pltpu.sync_copy(counts_vm, counts_hbm)


def _route_sc(weights):
    wp = jnp.concatenate(
        [weights, weights, weights, weights[:, :_L - 3 * _E]], axis=1)
    run = pl.kernel(
        _router_body,
        out_type=(jax.ShapeDtypeStruct((_B, _L), jnp.int32),
                  jax.ShapeDtypeStruct((_B, _L), jnp.float32),
                  jax.ShapeDtypeStruct((_L,), jnp.int32)),
        mesh=plsc.VectorSubcoreMesh(core_axis_name="c", subcore_axis_name="s"),
        compiler_params=pltpu.CompilerParams(needs_layout_passes=False),
        scratch_types=(pltpu.VMEM((_B, _L), jnp.float32),
                       pltpu.VMEM((_B, _L), jnp.int32),
                       pltpu.VMEM((_B, _L), jnp.float32),
                       pltpu.VMEM((_L,), jnp.int32),
                       pltpu.VMEM((2 * _L,), jnp.float32)),
    )
    topk_p, nw_p, counts_p = run(wp)
    return topk_p[:, :_K], nw_p[:, :_K], counts_p[:_E]


def kernel(feat, weights, w1, b1, wdw, bdw, wsca, bsca, w3, b3,
           ln1_g, ln1_b, w4, b4, w5, b5, ln2_g, ln2_b, beta, gamma):
    topk_i, nw, counts = _route_sc(weights)

    feat_r = feat.transpose(0, 2, 3, 1).reshape(_B, _HW, _C)
    bf = jnp.bfloat16
    tr = lambda m: m.transpose(0, 2, 1)
    row = lambda v: v.reshape(_E, 1, -1)
    w1g = (tr(w1) * ln1_g[:, :, None]).astype(bf)
    bb1 = row(b1 + jnp.einsum('ec,eoc->eo', ln1_b, w1)).astype(bf)
    w4g = (tr(w4) * ln2_g[:, :, None]).astype(bf)
    bb4 = row(b4 + jnp.einsum('ec,eoc->eo', ln2_b, w4)).astype(bf)
    wdw9 = wdw.reshape(_E, _C2, 9).transpose(0, 2, 1).astype(bf)
    out_r = _run_naf(
        topk_i, nw, feat_r, w1g, bb1, wdw9, row(bdw).astype(bf), tr(wsca),
        row(bsca), tr(w3).astype(bf), row(b3), w4g, bb4,
        tr(w5).astype(bf), row(b5), row(beta), row(gamma))
    out = out_r.reshape(_B, _H, _W, _C).transpose(0, 3, 1, 2)
    return (out, counts, weights)
